# baseline (speedup 1.0000x reference)
import functools

import jax
import jax.numpy as jnp
from jax import lax
from jax.experimental import pallas as pl
from jax.experimental.pallas import tpu as pltpu
from jax.experimental.pallas import tpu_sc as plsc


def _make_gather(V, D, BT, S):
    B = BT * S
    info = plsc.get_sparse_core_info()
    NC, NS = info.num_cores, info.num_subcores
    NW = NC * NS
    assert B % NW == 0
    b_per_w = B // NW
    assert S % b_per_w == 0
    w_per_row = S // b_per_w
    CHUNK = 32
    assert b_per_w % CHUNK == 0
    n_chunks = b_per_w // CHUNK
    assert n_chunks >= 16

    mesh = plsc.VectorSubcoreMesh(core_axis_name="c", subcore_axis_name="s")

    NBUF = 5

    @functools.partial(
        pl.kernel,
        mesh=mesh,
        out_type=jax.ShapeDtypeStruct((B, D), jnp.float32),
        scratch_types=[
            pltpu.VMEM((b_per_w,), jnp.int32),
        ]
        + [pltpu.VMEM((CHUNK, D), jnp.float32)] * NBUF
        + [pltpu.SemaphoreType.DMA] * (2 * NBUF),
    )
    def k(tok_hbm, table_hbm, out_hbm, idx_v, *rest):
        bufs = rest[:NBUF]
        gsems = rest[NBUF:2 * NBUF]
        wsems = rest[2 * NBUF:]
        wid = lax.axis_index("s") * NC + lax.axis_index("c")
        base = wid * b_per_w
        row = wid // w_per_row
        col = (wid % w_per_row) * b_per_w
        pltpu.sync_copy(tok_hbm.at[row, pl.ds(col, b_per_w)], idx_v)

        def gather_desc(c, par):
            idx_slice = idx_v.at[pl.ds(c * CHUNK, CHUNK)]
            return pltpu.make_async_copy(
                table_hbm.at[idx_slice], bufs[par], gsems[par])

        def write_desc(c, par):
            dst = out_hbm.at[pl.ds(base + c * CHUNK, CHUNK)]
            return pltpu.make_async_copy(bufs[par], dst, wsems[par])

        def pos(c, par, do_issue, do_waitw):
            gather_desc(c, par).wait()
            write_desc(c, par).start()
            if do_issue:
                if do_waitw:
                    write_desc(c - 1, (par + NBUF - 1) % NBUF).wait()
                gather_desc(c + NBUF - 1, (par + NBUF - 1) % NBUF).start()

        for c in range(NBUF - 1):
            gather_desc(c, c).start()
        pos(0, 0, True, False)

        pre = (n_chunks - NBUF) % NBUF
        for c in range(1, 1 + pre):
            pos(c, c % NBUF, True, True)

        def body(i, carry):
            for kk in range(NBUF):
                c = NBUF * i + 1 + pre + kk
                pos(c, (1 + pre + kk) % NBUF, True, True)
            return carry

        lax.fori_loop(0, (n_chunks - NBUF - pre) // NBUF, body, 0)

        for c in range(n_chunks - (NBUF - 1), n_chunks):
            pos(c, c % NBUF, False, False)
        for c in range(n_chunks - NBUF, n_chunks):
            write_desc(c, c % NBUF).wait()

    return k


def kernel(tokens, W_E):
    B_, S_ = tokens.shape
    V, D = W_E.shape
    out = _make_gather(V, D, B_, S_)(tokens.astype(jnp.int32), W_E)
    return out.reshape(B_, S_, D)


# speedup vs baseline: 1.0035x; 1.0035x over previous
import functools

import jax
import jax.numpy as jnp
from jax import lax
from jax.experimental import pallas as pl
from jax.experimental.pallas import tpu as pltpu
from jax.experimental.pallas import tpu_sc as plsc


def _make_gather(V, D, BT, S):
    B = BT * S
    info = plsc.get_sparse_core_info()
    NC, NS = info.num_cores, info.num_subcores
    NW = NC * NS
    assert B % NW == 0
    b_per_w = B // NW
    assert S % b_per_w == 0
    w_per_row = S // b_per_w
    CHUNK = 32
    assert b_per_w % CHUNK == 0
    n_chunks = b_per_w // CHUNK
    assert n_chunks >= 16

    mesh = plsc.VectorSubcoreMesh(core_axis_name="c", subcore_axis_name="s")

    NBUF = 4

    @functools.partial(
        pl.kernel,
        mesh=mesh,
        out_type=jax.ShapeDtypeStruct((B, D), jnp.float32),
        scratch_types=[
            pltpu.VMEM((b_per_w,), jnp.int32),
        ]
        + [pltpu.VMEM((CHUNK, D), jnp.float32)] * NBUF
        + [pltpu.SemaphoreType.DMA] * (2 * NBUF),
    )
    def k(tok_hbm, table_hbm, out_hbm, idx_v, *rest):
        bufs = rest[:NBUF]
        gsems = rest[NBUF:2 * NBUF]
        wsems = rest[2 * NBUF:]
        wid = lax.axis_index("s") * NC + lax.axis_index("c")
        base = wid * b_per_w
        row = wid // w_per_row
        col = (wid % w_per_row) * b_per_w
        pltpu.sync_copy(tok_hbm.at[row, pl.ds(col, b_per_w)], idx_v)

        def gather_desc(c, par):
            idx_slice = idx_v.at[pl.ds(c * CHUNK, CHUNK)]
            return pltpu.make_async_copy(
                table_hbm.at[idx_slice], bufs[par], gsems[par])

        def write_desc(c, par):
            dst = out_hbm.at[pl.ds(base + c * CHUNK, CHUNK)]
            return pltpu.make_async_copy(bufs[par], dst, wsems[par])

        def pos(c, par, do_issue, do_waitw):
            gather_desc(c, par).wait()
            write_desc(c, par).start()
            if do_issue:
                if do_waitw:
                    write_desc(c - 1, (par + NBUF - 1) % NBUF).wait()
                gather_desc(c + NBUF - 1, (par + NBUF - 1) % NBUF).start()

        for c in range(NBUF - 1):
            gather_desc(c, c).start()
        pos(0, 0, True, False)

        pre = (n_chunks - NBUF) % NBUF
        for c in range(1, 1 + pre):
            pos(c, c % NBUF, True, True)

        def body(i, carry):
            for kk in range(NBUF):
                c = NBUF * i + 1 + pre + kk
                pos(c, (1 + pre + kk) % NBUF, True, True)
            return carry

        lax.fori_loop(0, (n_chunks - NBUF - pre) // NBUF, body, 0)

        for c in range(n_chunks - (NBUF - 1), n_chunks):
            pos(c, c % NBUF, False, False)
        for c in range(n_chunks - NBUF, n_chunks):
            write_desc(c, c % NBUF).wait()

    return k


def kernel(tokens, W_E):
    B_, S_ = tokens.shape
    V, D = W_E.shape
    out = _make_gather(V, D, B_, S_)(tokens.astype(jnp.int32), W_E)
    return out.reshape(B_, S_, D)
